# Initial kernel scaffold; baseline (speedup 1.0000x reference)
import jax
import jax.numpy as jnp
from jax.experimental import pallas as pl


def kernel(x, w1, b1, w2, b2, gamma, beta, ws, bs, gamma_s, beta_s):
    raise NotImplementedError("write your pallas kernel here")


# speedup vs baseline: 10.6191x; 10.6191x over previous
import functools

import jax
import jax.numpy as jnp
from jax.experimental import pallas as pl
from jax.experimental.pallas import tpu as pltpu

EPS = 1e-5
VMEM_LIMIT = 48 * 1024 * 1024
CDT = jnp.bfloat16
F32 = jnp.float32


def _p1_kernel(xt_ref, wa_ref, wb_ref, ws_ref, y1_ref, r_ref, st_ref):
    cin = wa_ref.shape[0]
    l_out = xt_ref.shape[0]
    xt = xt_ref[...]
    prev = jnp.concatenate(
        [jnp.zeros((1, cin), xt.dtype), xt[: l_out - 1, cin:]], axis=0)
    y1 = jnp.dot(prev, wa_ref[...], preferred_element_type=F32)
    y1 = y1 + jnp.dot(xt, wb_ref[...], preferred_element_type=F32)
    r = jnp.dot(xt[:, :cin], ws_ref[...], preferred_element_type=F32)
    y1_ref[...] = y1.astype(y1_ref.dtype)
    r_ref[...] = r.astype(r_ref.dtype)
    st_ref[...] = jnp.concatenate(
        [jnp.sum(y1, axis=0, keepdims=True),
         jnp.sum(y1 * y1, axis=0, keepdims=True),
         jnp.sum(r, axis=0, keepdims=True),
         jnp.sum(r * r, axis=0, keepdims=True)], axis=0)


def _bn_affine(s, ssq, count, gamma, beta):
    mean = s * (1.0 / count)
    var = jnp.maximum(ssq * (1.0 / count) - mean * mean, 0.0)
    a = gamma * jax.lax.rsqrt(var + EPS)
    return a, beta - a * mean


def _p2_kernel(y1_ref, st1_ref, g_ref, bta_ref, w2_ref, y2_ref, st2_ref, hp_ref,
               *, count):
    k = w2_ref.shape[0]
    pad = k // 2
    l_out = y1_ref.shape[0]

    s1 = jnp.sum(st1_ref[...], axis=0)
    a1, b1 = _bn_affine(s1[0:1], s1[1:2], count, g_ref[...], bta_ref[...])
    h = jnp.maximum(a1 * y1_ref[...].astype(F32) + b1, 0.0)

    hp_ref[...] = jnp.zeros_like(hp_ref)
    hp_ref[pad:pad + l_out, :] = h.astype(hp_ref.dtype)

    y2 = jnp.dot(hp_ref[0:l_out, :], w2_ref[0], preferred_element_type=F32)
    for t in range(1, k):
        y2 = y2 + jnp.dot(hp_ref[t:t + l_out, :], w2_ref[t],
                          preferred_element_type=F32)
    y2_ref[...] = y2.astype(y2_ref.dtype)
    st2_ref[...] = jnp.concatenate(
        [jnp.sum(y2, axis=0, keepdims=True),
         jnp.sum(y2 * y2, axis=0, keepdims=True)], axis=0)


def _p3_kernel(y2_ref, r_ref, st1_ref, st2_ref, g_ref, bta_ref, gs_ref, bs_ref,
               out_ref, *, count):
    s1 = jnp.sum(st1_ref[...], axis=0)
    s2 = jnp.sum(st2_ref[...], axis=0)
    a2, b2 = _bn_affine(s2[0:1], s2[1:2], count, g_ref[...], bta_ref[...])
    a_s, b_s = _bn_affine(s1[2:3], s1[3:4], count, gs_ref[...], bs_ref[...])
    o = jnp.maximum(a2 * y2_ref[...].astype(F32) + b2
                    + a_s * r_ref[...].astype(F32) + b_s, 0.0)
    out_ref[...] = o.T


@jax.jit
def _run(x, w1, w2, gamma, beta, ws, gamma_s, beta_s):
    N, Cin, L = x.shape
    K = w1.shape[2]
    pad = K // 2
    Cout = w1.shape[0]
    Lout = L // 2
    C2 = 2 * Cin
    count = float(N * Lout)

    xt = jnp.transpose(x, (0, 2, 1)).reshape(N, Lout, C2).astype(CDT)

    w1t = jnp.transpose(w1, (2, 1, 0)).astype(CDT)
    wa = w1t[0]
    wb = jnp.concatenate([w1t[1], w1t[2]], axis=0)
    wsm = jnp.transpose(ws[:, :, 0], (1, 0)).astype(CDT)
    w2t = jnp.transpose(w2, (2, 1, 0)).astype(CDT)

    row = lambda v: v.astype(F32).reshape(1, Cout)
    g, bta = row(gamma), row(beta)
    gs, bs = row(gamma_s), row(beta_s)

    cparams = pltpu.CompilerParams(
        dimension_semantics=("parallel",), vmem_limit_bytes=VMEM_LIMIT)

    y1, r, st1 = pl.pallas_call(
        _p1_kernel,
        grid=(N,),
        in_specs=[
            pl.BlockSpec((None, Lout, C2), lambda n: (n, 0, 0)),
            pl.BlockSpec((Cin, Cout), lambda n: (0, 0)),
            pl.BlockSpec((C2, Cout), lambda n: (0, 0)),
            pl.BlockSpec((Cin, Cout), lambda n: (0, 0)),
        ],
        out_specs=[
            pl.BlockSpec((None, Lout, Cout), lambda n: (n, 0, 0)),
            pl.BlockSpec((None, Lout, Cout), lambda n: (n, 0, 0)),
            pl.BlockSpec((None, 4, Cout), lambda n: (n, 0, 0)),
        ],
        out_shape=[
            jax.ShapeDtypeStruct((N, Lout, Cout), CDT),
            jax.ShapeDtypeStruct((N, Lout, Cout), CDT),
            jax.ShapeDtypeStruct((N, 4, Cout), F32),
        ],
        compiler_params=cparams,
    )(xt, wa, wb, wsm)

    y2, st2 = pl.pallas_call(
        functools.partial(_p2_kernel, count=count),
        grid=(N,),
        in_specs=[
            pl.BlockSpec((None, Lout, Cout), lambda n: (n, 0, 0)),
            pl.BlockSpec((N, 4, Cout), lambda n: (0, 0, 0)),
            pl.BlockSpec((1, Cout), lambda n: (0, 0)),
            pl.BlockSpec((1, Cout), lambda n: (0, 0)),
            pl.BlockSpec((K, Cout, Cout), lambda n: (0, 0, 0)),
        ],
        out_specs=[
            pl.BlockSpec((None, Lout, Cout), lambda n: (n, 0, 0)),
            pl.BlockSpec((None, 2, Cout), lambda n: (n, 0, 0)),
        ],
        out_shape=[
            jax.ShapeDtypeStruct((N, Lout, Cout), CDT),
            jax.ShapeDtypeStruct((N, 2, Cout), F32),
        ],
        scratch_shapes=[pltpu.VMEM((Lout + 2 * pad, Cout), CDT)],
        compiler_params=cparams,
    )(y1, st1, g, bta, w2t)

    out = pl.pallas_call(
        functools.partial(_p3_kernel, count=count),
        grid=(N,),
        in_specs=[
            pl.BlockSpec((None, Lout, Cout), lambda n: (n, 0, 0)),
            pl.BlockSpec((None, Lout, Cout), lambda n: (n, 0, 0)),
            pl.BlockSpec((N, 4, Cout), lambda n: (0, 0, 0)),
            pl.BlockSpec((N, 2, Cout), lambda n: (0, 0, 0)),
            pl.BlockSpec((1, Cout), lambda n: (0, 0)),
            pl.BlockSpec((1, Cout), lambda n: (0, 0)),
            pl.BlockSpec((1, Cout), lambda n: (0, 0)),
            pl.BlockSpec((1, Cout), lambda n: (0, 0)),
        ],
        out_specs=pl.BlockSpec((None, Cout, Lout), lambda n: (n, 0, 0)),
        out_shape=jax.ShapeDtypeStruct((N, Cout, Lout), F32),
        compiler_params=cparams,
    )(y2, r, st1, st2, g, bta, gs, bs)

    return out


def kernel(x, w1, b1, w2, b2, gamma, beta, ws, bs, gamma_s, beta_s):
    return _run(x.astype(F32), w1, w2, gamma, beta, ws, gamma_s, beta_s)


# speedup vs baseline: 17.4473x; 1.6430x over previous
import functools

import jax
import jax.numpy as jnp
from jax.experimental import pallas as pl
from jax.experimental.pallas import tpu as pltpu

EPS = 1e-5
VMEM_LIMIT = 48 * 1024 * 1024
CDT = jnp.bfloat16
F32 = jnp.float32
BATCH = 8


def _sums(*arrs):
    return jnp.concatenate(
        [jnp.sum(a, axis=0, keepdims=True) for a in arrs], axis=0)


def _p1_kernel(xt_ref, wa_ref, wb_ref, ws_ref, y1_ref, r_ref, st_ref):
    b, l_out, c2 = xt_ref.shape
    cin = wa_ref.shape[0]
    xt3 = xt_ref[...]
    xt = xt3.reshape(b * l_out, c2)
    prev = jnp.concatenate(
        [jnp.zeros((b, 1, cin), xt.dtype), xt3[:, : l_out - 1, cin:]],
        axis=1).reshape(b * l_out, cin)
    y1 = jnp.dot(prev, wa_ref[...], preferred_element_type=F32)
    y1 = y1 + jnp.dot(xt, wb_ref[...], preferred_element_type=F32)
    r = jnp.dot(xt[:, :cin], ws_ref[...], preferred_element_type=F32)
    y1_ref[...] = y1.astype(y1_ref.dtype).reshape(b, l_out, -1)
    r_ref[...] = r.astype(r_ref.dtype).reshape(b, l_out, -1)
    st_ref[...] = _sums(y1, y1 * y1, r, r * r)


def _bn_affine(s, ssq, count, gamma, beta):
    mean = s * (1.0 / count)
    var = jnp.maximum(ssq * (1.0 / count) - mean * mean, 0.0)
    a = gamma * jax.lax.rsqrt(var + EPS)
    return a, beta - a * mean


def _p2_kernel(y1_ref, st1_ref, g_ref, bta_ref, w2_ref, y2_ref, st2_ref, *, count):
    b, l_out, c = y1_ref.shape

    s1 = jnp.sum(st1_ref[...], axis=0)
    a1, b1 = _bn_affine(s1[0:1], s1[1:2], count, g_ref[...], bta_ref[...])
    h3 = jnp.maximum(
        a1 * y1_ref[...].astype(F32).reshape(b * l_out, c) + b1,
        0.0).astype(CDT).reshape(b, l_out, c)

    zrow = jnp.zeros((b, 1, c), CDT)
    h_m = h3.reshape(b * l_out, c)
    h_r = jnp.concatenate([zrow, h3[:, : l_out - 1, :]],
                          axis=1).reshape(b * l_out, c)
    h_l = jnp.concatenate([h3[:, 1:, :], zrow],
                          axis=1).reshape(b * l_out, c)
    y2 = jnp.dot(h_r, w2_ref[0], preferred_element_type=F32)
    y2 = y2 + jnp.dot(h_m, w2_ref[1], preferred_element_type=F32)
    y2 = y2 + jnp.dot(h_l, w2_ref[2], preferred_element_type=F32)
    y2_ref[...] = y2.astype(y2_ref.dtype).reshape(b, l_out, c)
    st2_ref[...] = _sums(y2, y2 * y2)


def _p3_kernel(y2_ref, r_ref, st1_ref, st2_ref, g_ref, bta_ref, gs_ref, bs_ref,
               out_ref, *, count):
    b = y2_ref.shape[0]
    s1 = jnp.sum(st1_ref[...], axis=0)
    s2 = jnp.sum(st2_ref[...], axis=0)
    a2, b2 = _bn_affine(s2[0:1], s2[1:2], count, g_ref[...], bta_ref[...])
    a_s, b_s = _bn_affine(s1[2:3], s1[3:4], count, gs_ref[...], bs_ref[...])
    for i in range(b):
        o = jnp.maximum(a2 * y2_ref[i].astype(F32) + b2
                        + a_s * r_ref[i].astype(F32) + b_s, 0.0)
        out_ref[i] = o.T


@jax.jit
def _run(x, w1, w2, gamma, beta, ws, gamma_s, beta_s):
    N, Cin, L = x.shape
    K = w1.shape[2]
    Cout = w1.shape[0]
    Lout = L // 2
    C2 = 2 * Cin
    count = float(N * Lout)
    B = BATCH
    G = N // B

    xt = jnp.transpose(x, (0, 2, 1)).reshape(N, Lout, C2).astype(CDT)

    w1t = jnp.transpose(w1, (2, 1, 0)).astype(CDT)
    wa = w1t[0]
    wb = jnp.concatenate([w1t[1], w1t[2]], axis=0)
    wsm = jnp.transpose(ws[:, :, 0], (1, 0)).astype(CDT)
    w2t = jnp.transpose(w2, (2, 1, 0)).astype(CDT)

    row = lambda v: v.astype(F32).reshape(1, Cout)
    g, bta = row(gamma), row(beta)
    gs, bs = row(gamma_s), row(beta_s)

    cparams = pltpu.CompilerParams(
        dimension_semantics=("parallel",), vmem_limit_bytes=VMEM_LIMIT)

    y1, r, st1 = pl.pallas_call(
        _p1_kernel,
        grid=(G,),
        in_specs=[
            pl.BlockSpec((B, Lout, C2), lambda n: (n, 0, 0)),
            pl.BlockSpec((Cin, Cout), lambda n: (0, 0)),
            pl.BlockSpec((C2, Cout), lambda n: (0, 0)),
            pl.BlockSpec((Cin, Cout), lambda n: (0, 0)),
        ],
        out_specs=[
            pl.BlockSpec((B, Lout, Cout), lambda n: (n, 0, 0)),
            pl.BlockSpec((B, Lout, Cout), lambda n: (n, 0, 0)),
            pl.BlockSpec((None, 4, Cout), lambda n: (n, 0, 0)),
        ],
        out_shape=[
            jax.ShapeDtypeStruct((N, Lout, Cout), CDT),
            jax.ShapeDtypeStruct((N, Lout, Cout), CDT),
            jax.ShapeDtypeStruct((G, 4, Cout), F32),
        ],
        compiler_params=cparams,
    )(xt, wa, wb, wsm)

    y2, st2 = pl.pallas_call(
        functools.partial(_p2_kernel, count=count),
        grid=(G,),
        in_specs=[
            pl.BlockSpec((B, Lout, Cout), lambda n: (n, 0, 0)),
            pl.BlockSpec((G, 4, Cout), lambda n: (0, 0, 0)),
            pl.BlockSpec((1, Cout), lambda n: (0, 0)),
            pl.BlockSpec((1, Cout), lambda n: (0, 0)),
            pl.BlockSpec((K, Cout, Cout), lambda n: (0, 0, 0)),
        ],
        out_specs=[
            pl.BlockSpec((B, Lout, Cout), lambda n: (n, 0, 0)),
            pl.BlockSpec((None, 2, Cout), lambda n: (n, 0, 0)),
        ],
        out_shape=[
            jax.ShapeDtypeStruct((N, Lout, Cout), CDT),
            jax.ShapeDtypeStruct((G, 2, Cout), F32),
        ],
        compiler_params=cparams,
    )(y1, st1, g, bta, w2t)

    out = pl.pallas_call(
        functools.partial(_p3_kernel, count=count),
        grid=(G,),
        in_specs=[
            pl.BlockSpec((B, Lout, Cout), lambda n: (n, 0, 0)),
            pl.BlockSpec((B, Lout, Cout), lambda n: (n, 0, 0)),
            pl.BlockSpec((G, 4, Cout), lambda n: (0, 0, 0)),
            pl.BlockSpec((G, 2, Cout), lambda n: (0, 0, 0)),
            pl.BlockSpec((1, Cout), lambda n: (0, 0)),
            pl.BlockSpec((1, Cout), lambda n: (0, 0)),
            pl.BlockSpec((1, Cout), lambda n: (0, 0)),
            pl.BlockSpec((1, Cout), lambda n: (0, 0)),
        ],
        out_specs=pl.BlockSpec((B, Cout, Lout), lambda n: (n, 0, 0)),
        out_shape=jax.ShapeDtypeStruct((N, Cout, Lout), F32),
        compiler_params=cparams,
    )(y2, r, st1, st2, g, bta, gs, bs)

    return out


def kernel(x, w1, b1, w2, b2, gamma, beta, ws, bs, gamma_s, beta_s):
    return _run(x.astype(F32), w1, w2, gamma, beta, ws, gamma_s, beta_s)


# speedup vs baseline: 32.3450x; 1.8539x over previous
import functools

import jax
import jax.numpy as jnp
from jax.experimental import pallas as pl
from jax.experimental.pallas import tpu as pltpu

EPS = 1e-5
VMEM_LIMIT = 48 * 1024 * 1024
CDT = jnp.bfloat16
F32 = jnp.float32
BATCH = 8


def _sums(*arrs):
    return jnp.concatenate(
        [jnp.sum(a, axis=0, keepdims=True) for a in arrs], axis=0)


def _p1_kernel(x_ref, wa_ref, wb_ref, wc_ref, ws_ref, y1_ref, r_ref, st_ref,
               xts_ref):
    b, cin, l_in = x_ref.shape
    l_out = l_in // 2
    sy = ssy = sr = ssr = 0.0
    for i in range(b):
        xts_ref[...] = x_ref[i].T
        xev = xts_ref[0::2, :].astype(CDT)
        xod = xts_ref[1::2, :].astype(CDT)
        prev = jnp.concatenate(
            [jnp.zeros((1, cin), CDT), xod[: l_out - 1, :]], axis=0)
        y1 = jnp.dot(prev, wa_ref[...], preferred_element_type=F32)
        y1 = y1 + jnp.dot(xev, wb_ref[...], preferred_element_type=F32)
        y1 = y1 + jnp.dot(xod, wc_ref[...], preferred_element_type=F32)
        r = jnp.dot(xev, ws_ref[...], preferred_element_type=F32)
        y1_ref[i] = y1.astype(y1_ref.dtype)
        r_ref[i] = r.astype(r_ref.dtype)
        sy = sy + jnp.sum(y1, axis=0, keepdims=True)
        ssy = ssy + jnp.sum(y1 * y1, axis=0, keepdims=True)
        sr = sr + jnp.sum(r, axis=0, keepdims=True)
        ssr = ssr + jnp.sum(r * r, axis=0, keepdims=True)
    st_ref[...] = jnp.concatenate([sy, ssy, sr, ssr], axis=0)


def _bn_affine(s, ssq, count, gamma, beta):
    mean = s * (1.0 / count)
    var = jnp.maximum(ssq * (1.0 / count) - mean * mean, 0.0)
    a = gamma * jax.lax.rsqrt(var + EPS)
    return a, beta - a * mean


def _p2_kernel(y1_ref, st1_ref, g_ref, bta_ref, w2_ref, y2_ref, st2_ref, *, count):
    b, l_out, c = y1_ref.shape

    s1 = jnp.sum(st1_ref[...], axis=0)
    a1, b1 = _bn_affine(s1[0:1], s1[1:2], count, g_ref[...], bta_ref[...])
    h3 = jnp.maximum(
        a1 * y1_ref[...].astype(F32).reshape(b * l_out, c) + b1,
        0.0).astype(CDT).reshape(b, l_out, c)

    zrow = jnp.zeros((b, 1, c), CDT)
    h_m = h3.reshape(b * l_out, c)
    h_r = jnp.concatenate([zrow, h3[:, : l_out - 1, :]],
                          axis=1).reshape(b * l_out, c)
    h_l = jnp.concatenate([h3[:, 1:, :], zrow],
                          axis=1).reshape(b * l_out, c)
    y2 = jnp.dot(h_r, w2_ref[0], preferred_element_type=F32)
    y2 = y2 + jnp.dot(h_m, w2_ref[1], preferred_element_type=F32)
    y2 = y2 + jnp.dot(h_l, w2_ref[2], preferred_element_type=F32)
    y2_ref[...] = y2.astype(y2_ref.dtype).reshape(b, l_out, c)
    st2_ref[...] = _sums(y2, y2 * y2)


def _p3_kernel(y2_ref, r_ref, st1_ref, st2_ref, g_ref, bta_ref, gs_ref, bs_ref,
               out_ref, *, count):
    b = y2_ref.shape[0]
    s1 = jnp.sum(st1_ref[...], axis=0)
    s2 = jnp.sum(st2_ref[...], axis=0)
    a2, b2 = _bn_affine(s2[0:1], s2[1:2], count, g_ref[...], bta_ref[...])
    a_s, b_s = _bn_affine(s1[2:3], s1[3:4], count, gs_ref[...], bs_ref[...])
    for i in range(b):
        o = jnp.maximum(a2 * y2_ref[i].astype(F32) + b2
                        + a_s * r_ref[i].astype(F32) + b_s, 0.0)
        out_ref[i] = o.T


@jax.jit
def _run(x, w1, w2, gamma, beta, ws, gamma_s, beta_s):
    N, Cin, L = x.shape
    K = w1.shape[2]
    Cout = w1.shape[0]
    Lout = L // 2
    C2 = 2 * Cin
    count = float(N * Lout)
    B = BATCH
    G = N // B

    w1t = jnp.transpose(w1, (2, 1, 0)).astype(CDT)
    wa, wb, wc = w1t[0], w1t[1], w1t[2]
    wsm = jnp.transpose(ws[:, :, 0], (1, 0)).astype(CDT)
    w2t = jnp.transpose(w2, (2, 1, 0)).astype(CDT)

    row = lambda v: v.astype(F32).reshape(1, Cout)
    g, bta = row(gamma), row(beta)
    gs, bs = row(gamma_s), row(beta_s)

    cparams = pltpu.CompilerParams(
        dimension_semantics=("parallel",), vmem_limit_bytes=VMEM_LIMIT)

    y1, r, st1 = pl.pallas_call(
        _p1_kernel,
        grid=(G,),
        in_specs=[
            pl.BlockSpec((B, Cin, L), lambda n: (n, 0, 0)),
            pl.BlockSpec((Cin, Cout), lambda n: (0, 0)),
            pl.BlockSpec((Cin, Cout), lambda n: (0, 0)),
            pl.BlockSpec((Cin, Cout), lambda n: (0, 0)),
            pl.BlockSpec((Cin, Cout), lambda n: (0, 0)),
        ],
        out_specs=[
            pl.BlockSpec((B, Lout, Cout), lambda n: (n, 0, 0)),
            pl.BlockSpec((B, Lout, Cout), lambda n: (n, 0, 0)),
            pl.BlockSpec((None, 4, Cout), lambda n: (n, 0, 0)),
        ],
        out_shape=[
            jax.ShapeDtypeStruct((N, Lout, Cout), CDT),
            jax.ShapeDtypeStruct((N, Lout, Cout), CDT),
            jax.ShapeDtypeStruct((G, 4, Cout), F32),
        ],
        scratch_shapes=[pltpu.VMEM((L, Cin), F32)],
        compiler_params=cparams,
    )(x, wa, wb, wc, wsm)

    y2, st2 = pl.pallas_call(
        functools.partial(_p2_kernel, count=count),
        grid=(G,),
        in_specs=[
            pl.BlockSpec((B, Lout, Cout), lambda n: (n, 0, 0)),
            pl.BlockSpec((G, 4, Cout), lambda n: (0, 0, 0)),
            pl.BlockSpec((1, Cout), lambda n: (0, 0)),
            pl.BlockSpec((1, Cout), lambda n: (0, 0)),
            pl.BlockSpec((K, Cout, Cout), lambda n: (0, 0, 0)),
        ],
        out_specs=[
            pl.BlockSpec((B, Lout, Cout), lambda n: (n, 0, 0)),
            pl.BlockSpec((None, 2, Cout), lambda n: (n, 0, 0)),
        ],
        out_shape=[
            jax.ShapeDtypeStruct((N, Lout, Cout), CDT),
            jax.ShapeDtypeStruct((G, 2, Cout), F32),
        ],
        compiler_params=cparams,
    )(y1, st1, g, bta, w2t)

    out = pl.pallas_call(
        functools.partial(_p3_kernel, count=count),
        grid=(G,),
        in_specs=[
            pl.BlockSpec((B, Lout, Cout), lambda n: (n, 0, 0)),
            pl.BlockSpec((B, Lout, Cout), lambda n: (n, 0, 0)),
            pl.BlockSpec((G, 4, Cout), lambda n: (0, 0, 0)),
            pl.BlockSpec((G, 2, Cout), lambda n: (0, 0, 0)),
            pl.BlockSpec((1, Cout), lambda n: (0, 0)),
            pl.BlockSpec((1, Cout), lambda n: (0, 0)),
            pl.BlockSpec((1, Cout), lambda n: (0, 0)),
            pl.BlockSpec((1, Cout), lambda n: (0, 0)),
        ],
        out_specs=pl.BlockSpec((B, Cout, Lout), lambda n: (n, 0, 0)),
        out_shape=jax.ShapeDtypeStruct((N, Cout, Lout), F32),
        compiler_params=cparams,
    )(y2, r, st1, st2, g, bta, gs, bs)

    return out


def kernel(x, w1, b1, w2, b2, gamma, beta, ws, bs, gamma_s, beta_s):
    return _run(x.astype(F32), w1, w2, gamma, beta, ws, gamma_s, beta_s)


# speedup vs baseline: 32.6306x; 1.0088x over previous
import functools

import jax
import jax.numpy as jnp
from jax.experimental import pallas as pl
from jax.experimental.pallas import tpu as pltpu

EPS = 1e-5
VMEM_LIMIT = 48 * 1024 * 1024
CDT = jnp.bfloat16
F32 = jnp.float32
BATCH = 8


def _sums(*arrs):
    return jnp.concatenate(
        [jnp.sum(a, axis=0, keepdims=True) for a in arrs], axis=0)


def _p1_kernel(x_ref, wa_ref, wb_ref, wc_ref, ws_ref, y1_ref, r_ref, st_ref,
               xts_ref):
    b, cin, l_in = x_ref.shape
    l_out = l_in // 2
    sy = ssy = sr = ssr = 0.0
    for i in range(b):
        xts_ref[...] = x_ref[i].T
        xev = xts_ref[0::2, :].astype(CDT)
        xod = xts_ref[1::2, :].astype(CDT)
        prev = jnp.concatenate(
            [jnp.zeros((1, cin), CDT), xod[: l_out - 1, :]], axis=0)
        y1 = jnp.dot(prev, wa_ref[...], preferred_element_type=F32)
        y1 = y1 + jnp.dot(xev, wb_ref[...], preferred_element_type=F32)
        y1 = y1 + jnp.dot(xod, wc_ref[...], preferred_element_type=F32)
        r = jnp.dot(xev, ws_ref[...], preferred_element_type=F32)
        y1_ref[i] = y1.astype(y1_ref.dtype)
        r_ref[i] = r.astype(r_ref.dtype)
        sy = sy + jnp.sum(y1, axis=0, keepdims=True)
        ssy = ssy + jnp.sum(y1 * y1, axis=0, keepdims=True)
        sr = sr + jnp.sum(r, axis=0, keepdims=True)
        ssr = ssr + jnp.sum(r * r, axis=0, keepdims=True)
    st_ref[...] = jnp.concatenate([sy, ssy, sr, ssr], axis=0)


def _bn_affine(s, ssq, count, gamma, beta):
    mean = s * (1.0 / count)
    var = jnp.maximum(ssq * (1.0 / count) - mean * mean, 0.0)
    a = gamma * jax.lax.rsqrt(var + EPS)
    return a, beta - a * mean


def _p2_kernel(y1_ref, st1_ref, g_ref, bta_ref, w2_ref, y2_ref, st2_ref, *, count):
    b, l_out, c = y1_ref.shape

    s1 = jnp.sum(st1_ref[...], axis=0)
    a1, b1 = _bn_affine(s1[0:1], s1[1:2], count, g_ref[...], bta_ref[...])
    a1c, b1c = a1.astype(CDT), b1.astype(CDT)
    h3 = jnp.maximum(a1c * y1_ref[...].reshape(b * l_out, c) + b1c,
                     jnp.zeros((), CDT)).reshape(b, l_out, c)

    zrow = jnp.zeros((b, 1, c), CDT)
    h_m = h3.reshape(b * l_out, c)
    h_r = jnp.concatenate([zrow, h3[:, : l_out - 1, :]],
                          axis=1).reshape(b * l_out, c)
    h_l = jnp.concatenate([h3[:, 1:, :], zrow],
                          axis=1).reshape(b * l_out, c)
    y2 = jnp.dot(h_r, w2_ref[0], preferred_element_type=F32)
    y2 = y2 + jnp.dot(h_m, w2_ref[1], preferred_element_type=F32)
    y2 = y2 + jnp.dot(h_l, w2_ref[2], preferred_element_type=F32)
    y2_ref[...] = y2.astype(y2_ref.dtype).reshape(b, l_out, c)
    st2_ref[...] = _sums(y2, y2 * y2)


def _p3_kernel(y2_ref, r_ref, st1_ref, st2_ref, g_ref, bta_ref, gs_ref, bs_ref,
               out_ref, *, count):
    b = y2_ref.shape[0]
    s1 = jnp.sum(st1_ref[...], axis=0)
    s2 = jnp.sum(st2_ref[...], axis=0)
    a2, b2 = _bn_affine(s2[0:1], s2[1:2], count, g_ref[...], bta_ref[...])
    a_s, b_s = _bn_affine(s1[2:3], s1[3:4], count, gs_ref[...], bs_ref[...])
    for i in range(b):
        o = jnp.maximum(a2 * y2_ref[i].astype(F32) + b2
                        + a_s * r_ref[i].astype(F32) + b_s, 0.0)
        out_ref[i] = o.T


@jax.jit
def _run(x, w1, w2, gamma, beta, ws, gamma_s, beta_s):
    N, Cin, L = x.shape
    K = w1.shape[2]
    Cout = w1.shape[0]
    Lout = L // 2
    C2 = 2 * Cin
    count = float(N * Lout)
    B = BATCH
    G = N // B

    w1t = jnp.transpose(w1, (2, 1, 0)).astype(CDT)
    wa, wb, wc = w1t[0], w1t[1], w1t[2]
    wsm = jnp.transpose(ws[:, :, 0], (1, 0)).astype(CDT)
    w2t = jnp.transpose(w2, (2, 1, 0)).astype(CDT)

    row = lambda v: v.astype(F32).reshape(1, Cout)
    g, bta = row(gamma), row(beta)
    gs, bs = row(gamma_s), row(beta_s)

    cparams = pltpu.CompilerParams(
        dimension_semantics=("parallel",), vmem_limit_bytes=VMEM_LIMIT)

    y1, r, st1 = pl.pallas_call(
        _p1_kernel,
        grid=(G,),
        in_specs=[
            pl.BlockSpec((B, Cin, L), lambda n: (n, 0, 0)),
            pl.BlockSpec((Cin, Cout), lambda n: (0, 0)),
            pl.BlockSpec((Cin, Cout), lambda n: (0, 0)),
            pl.BlockSpec((Cin, Cout), lambda n: (0, 0)),
            pl.BlockSpec((Cin, Cout), lambda n: (0, 0)),
        ],
        out_specs=[
            pl.BlockSpec((B, Lout, Cout), lambda n: (n, 0, 0)),
            pl.BlockSpec((B, Lout, Cout), lambda n: (n, 0, 0)),
            pl.BlockSpec((None, 4, Cout), lambda n: (n, 0, 0)),
        ],
        out_shape=[
            jax.ShapeDtypeStruct((N, Lout, Cout), CDT),
            jax.ShapeDtypeStruct((N, Lout, Cout), CDT),
            jax.ShapeDtypeStruct((G, 4, Cout), F32),
        ],
        scratch_shapes=[pltpu.VMEM((L, Cin), F32)],
        compiler_params=cparams,
    )(x, wa, wb, wc, wsm)

    y2, st2 = pl.pallas_call(
        functools.partial(_p2_kernel, count=count),
        grid=(G,),
        in_specs=[
            pl.BlockSpec((B, Lout, Cout), lambda n: (n, 0, 0)),
            pl.BlockSpec((G, 4, Cout), lambda n: (0, 0, 0)),
            pl.BlockSpec((1, Cout), lambda n: (0, 0)),
            pl.BlockSpec((1, Cout), lambda n: (0, 0)),
            pl.BlockSpec((K, Cout, Cout), lambda n: (0, 0, 0)),
        ],
        out_specs=[
            pl.BlockSpec((B, Lout, Cout), lambda n: (n, 0, 0)),
            pl.BlockSpec((None, 2, Cout), lambda n: (n, 0, 0)),
        ],
        out_shape=[
            jax.ShapeDtypeStruct((N, Lout, Cout), CDT),
            jax.ShapeDtypeStruct((G, 2, Cout), F32),
        ],
        compiler_params=cparams,
    )(y1, st1, g, bta, w2t)

    out = pl.pallas_call(
        functools.partial(_p3_kernel, count=count),
        grid=(G,),
        in_specs=[
            pl.BlockSpec((B, Lout, Cout), lambda n: (n, 0, 0)),
            pl.BlockSpec((B, Lout, Cout), lambda n: (n, 0, 0)),
            pl.BlockSpec((G, 4, Cout), lambda n: (0, 0, 0)),
            pl.BlockSpec((G, 2, Cout), lambda n: (0, 0, 0)),
            pl.BlockSpec((1, Cout), lambda n: (0, 0)),
            pl.BlockSpec((1, Cout), lambda n: (0, 0)),
            pl.BlockSpec((1, Cout), lambda n: (0, 0)),
            pl.BlockSpec((1, Cout), lambda n: (0, 0)),
        ],
        out_specs=pl.BlockSpec((B, Cout, Lout), lambda n: (n, 0, 0)),
        out_shape=jax.ShapeDtypeStruct((N, Cout, Lout), F32),
        compiler_params=cparams,
    )(y2, r, st1, st2, g, bta, gs, bs)

    return out


def kernel(x, w1, b1, w2, b2, gamma, beta, ws, bs, gamma_s, beta_s):
    return _run(x.astype(F32), w1, w2, gamma, beta, ws, gamma_s, beta_s)


# speedup vs baseline: 33.4716x; 1.0258x over previous
import functools

import jax
import jax.numpy as jnp
from jax.experimental import pallas as pl
from jax.experimental.pallas import tpu as pltpu

EPS = 1e-5
VMEM_LIMIT = 48 * 1024 * 1024
CDT = jnp.bfloat16
F32 = jnp.float32
BATCH = 16


def _sums(*arrs):
    return jnp.concatenate(
        [jnp.sum(a, axis=0, keepdims=True) for a in arrs], axis=0)


def _p1_kernel(x_ref, wa_ref, wb_ref, wc_ref, ws_ref, y1_ref, r_ref, st_ref,
               xts_ref):
    b, cin, l_in = x_ref.shape
    l_out = l_in // 2
    sy = ssy = sr = ssr = 0.0
    for i in range(b):
        xts_ref[...] = x_ref[i].T
        xev = xts_ref[0::2, :].astype(CDT)
        xod = xts_ref[1::2, :].astype(CDT)
        prev = jnp.concatenate(
            [jnp.zeros((1, cin), CDT), xod[: l_out - 1, :]], axis=0)
        y1 = jnp.dot(prev, wa_ref[...], preferred_element_type=F32)
        y1 = y1 + jnp.dot(xev, wb_ref[...], preferred_element_type=F32)
        y1 = y1 + jnp.dot(xod, wc_ref[...], preferred_element_type=F32)
        r = jnp.dot(xev, ws_ref[...], preferred_element_type=F32)
        y1_ref[i] = y1.astype(y1_ref.dtype)
        r_ref[i] = r.astype(r_ref.dtype)
        sy = sy + jnp.sum(y1, axis=0, keepdims=True)
        ssy = ssy + jnp.sum(y1 * y1, axis=0, keepdims=True)
        sr = sr + jnp.sum(r, axis=0, keepdims=True)
        ssr = ssr + jnp.sum(r * r, axis=0, keepdims=True)
    st_ref[...] = jnp.concatenate([sy, ssy, sr, ssr], axis=0)


def _bn_affine(s, ssq, count, gamma, beta):
    mean = s * (1.0 / count)
    var = jnp.maximum(ssq * (1.0 / count) - mean * mean, 0.0)
    a = gamma * jax.lax.rsqrt(var + EPS)
    return a, beta - a * mean


def _p2_kernel(y1_ref, st1_ref, g_ref, bta_ref, w2_ref, y2_ref, st2_ref, *, count):
    b, l_out, c = y1_ref.shape

    s1 = jnp.sum(st1_ref[...], axis=0)
    a1, b1 = _bn_affine(s1[0:1], s1[1:2], count, g_ref[...], bta_ref[...])
    a1c, b1c = a1.astype(CDT), b1.astype(CDT)
    h3 = jnp.maximum(a1c * y1_ref[...].reshape(b * l_out, c) + b1c,
                     jnp.zeros((), CDT)).reshape(b, l_out, c)

    zrow = jnp.zeros((b, 1, c), CDT)
    h_m = h3.reshape(b * l_out, c)
    h_r = jnp.concatenate([zrow, h3[:, : l_out - 1, :]],
                          axis=1).reshape(b * l_out, c)
    h_l = jnp.concatenate([h3[:, 1:, :], zrow],
                          axis=1).reshape(b * l_out, c)
    y2 = jnp.dot(h_r, w2_ref[0], preferred_element_type=F32)
    y2 = y2 + jnp.dot(h_m, w2_ref[1], preferred_element_type=F32)
    y2 = y2 + jnp.dot(h_l, w2_ref[2], preferred_element_type=F32)
    y2_ref[...] = y2.astype(y2_ref.dtype).reshape(b, l_out, c)
    st2_ref[...] = _sums(y2, y2 * y2)


def _p3_kernel(y2_ref, r_ref, st1_ref, st2_ref, g_ref, bta_ref, gs_ref, bs_ref,
               out_ref, *, count):
    b = y2_ref.shape[0]
    s1 = jnp.sum(st1_ref[...], axis=0)
    s2 = jnp.sum(st2_ref[...], axis=0)
    a2, b2 = _bn_affine(s2[0:1], s2[1:2], count, g_ref[...], bta_ref[...])
    a_s, b_s = _bn_affine(s1[2:3], s1[3:4], count, gs_ref[...], bs_ref[...])
    for i in range(b):
        o = jnp.maximum(a2 * y2_ref[i].astype(F32) + b2
                        + a_s * r_ref[i].astype(F32) + b_s, 0.0)
        out_ref[i] = o.T


@jax.jit
def _run(x, w1, w2, gamma, beta, ws, gamma_s, beta_s):
    N, Cin, L = x.shape
    K = w1.shape[2]
    Cout = w1.shape[0]
    Lout = L // 2
    C2 = 2 * Cin
    count = float(N * Lout)
    B = BATCH
    G = N // B

    w1t = jnp.transpose(w1, (2, 1, 0)).astype(CDT)
    wa, wb, wc = w1t[0], w1t[1], w1t[2]
    wsm = jnp.transpose(ws[:, :, 0], (1, 0)).astype(CDT)
    w2t = jnp.transpose(w2, (2, 1, 0)).astype(CDT)

    row = lambda v: v.astype(F32).reshape(1, Cout)
    g, bta = row(gamma), row(beta)
    gs, bs = row(gamma_s), row(beta_s)

    cparams = pltpu.CompilerParams(
        dimension_semantics=("parallel",), vmem_limit_bytes=VMEM_LIMIT)

    y1, r, st1 = pl.pallas_call(
        _p1_kernel,
        grid=(G,),
        in_specs=[
            pl.BlockSpec((B, Cin, L), lambda n: (n, 0, 0)),
            pl.BlockSpec((Cin, Cout), lambda n: (0, 0)),
            pl.BlockSpec((Cin, Cout), lambda n: (0, 0)),
            pl.BlockSpec((Cin, Cout), lambda n: (0, 0)),
            pl.BlockSpec((Cin, Cout), lambda n: (0, 0)),
        ],
        out_specs=[
            pl.BlockSpec((B, Lout, Cout), lambda n: (n, 0, 0)),
            pl.BlockSpec((B, Lout, Cout), lambda n: (n, 0, 0)),
            pl.BlockSpec((None, 4, Cout), lambda n: (n, 0, 0)),
        ],
        out_shape=[
            jax.ShapeDtypeStruct((N, Lout, Cout), CDT),
            jax.ShapeDtypeStruct((N, Lout, Cout), CDT),
            jax.ShapeDtypeStruct((G, 4, Cout), F32),
        ],
        scratch_shapes=[pltpu.VMEM((L, Cin), F32)],
        compiler_params=cparams,
    )(x, wa, wb, wc, wsm)

    y2, st2 = pl.pallas_call(
        functools.partial(_p2_kernel, count=count),
        grid=(G,),
        in_specs=[
            pl.BlockSpec((B, Lout, Cout), lambda n: (n, 0, 0)),
            pl.BlockSpec((G, 4, Cout), lambda n: (0, 0, 0)),
            pl.BlockSpec((1, Cout), lambda n: (0, 0)),
            pl.BlockSpec((1, Cout), lambda n: (0, 0)),
            pl.BlockSpec((K, Cout, Cout), lambda n: (0, 0, 0)),
        ],
        out_specs=[
            pl.BlockSpec((B, Lout, Cout), lambda n: (n, 0, 0)),
            pl.BlockSpec((None, 2, Cout), lambda n: (n, 0, 0)),
        ],
        out_shape=[
            jax.ShapeDtypeStruct((N, Lout, Cout), CDT),
            jax.ShapeDtypeStruct((G, 2, Cout), F32),
        ],
        compiler_params=cparams,
    )(y1, st1, g, bta, w2t)

    out = pl.pallas_call(
        functools.partial(_p3_kernel, count=count),
        grid=(G,),
        in_specs=[
            pl.BlockSpec((B, Lout, Cout), lambda n: (n, 0, 0)),
            pl.BlockSpec((B, Lout, Cout), lambda n: (n, 0, 0)),
            pl.BlockSpec((G, 4, Cout), lambda n: (0, 0, 0)),
            pl.BlockSpec((G, 2, Cout), lambda n: (0, 0, 0)),
            pl.BlockSpec((1, Cout), lambda n: (0, 0)),
            pl.BlockSpec((1, Cout), lambda n: (0, 0)),
            pl.BlockSpec((1, Cout), lambda n: (0, 0)),
            pl.BlockSpec((1, Cout), lambda n: (0, 0)),
        ],
        out_specs=pl.BlockSpec((B, Cout, Lout), lambda n: (n, 0, 0)),
        out_shape=jax.ShapeDtypeStruct((N, Cout, Lout), F32),
        compiler_params=cparams,
    )(y2, r, st1, st2, g, bta, gs, bs)

    return out


def kernel(x, w1, b1, w2, b2, gamma, beta, ws, bs, gamma_s, beta_s):
    return _run(x.astype(F32), w1, w2, gamma, beta, ws, gamma_s, beta_s)
